# Initial kernel scaffold; baseline (speedup 1.0000x reference)
import jax
import jax.numpy as jnp
from jax.experimental import pallas as pl


def kernel(nodes, edges, globals_, senders, receivers, params, deterministic):
    raise NotImplementedError("write your pallas kernel here")


# speedup vs baseline: 1.9463x; 1.9463x over previous
import functools

import jax
import jax.numpy as jnp
from jax import lax
from jax.experimental import pallas as pl
from jax.experimental.pallas import tpu as pltpu
from jax.experimental.pallas import tpu_sc as plsc

_N = 10000
_E = 320000
_L = 64

_NC = 2
_NS = 16
_NW = _NC * _NS
_CH = 128
_NCHUNKS = _E // _CH
_FULL = _NCHUNKS // _NW
_REM = _NCHUNKS - _FULL * _NW
_RPS = _N // _NS
_ZR = 125

_BLKE = 2560
_BLKN = 2000

_EPS = 1e-6


def _layer_norm(x, gam, bet):
    mu = jnp.mean(x, axis=-1, keepdims=True)
    var = jnp.mean(jnp.square(x - mu), axis=-1, keepdims=True)
    return gam * ((x - mu) / jnp.sqrt(var + _EPS)) + bet


def _mlp3_body(x_ref, w1, b1, w2, b2, w3, b3, o_ref):
    h = jax.nn.gelu(x_ref[...] @ w1[...] + b1[...])
    h = jax.nn.gelu(h @ w2[...] + b2[...])
    o_ref[...] = h @ w3[...] + b3[...]


def _mlp3(x, p, blk):
    (w1, b1), (w2, b2), (w3, b3) = p
    m, d = x.shape
    dout = w3.shape[1]
    full = lambda a: pl.BlockSpec(a.shape, lambda i: (0,) * a.ndim)
    b1, b2, b3 = b1.reshape(1, -1), b2.reshape(1, -1), b3.reshape(1, -1)
    return pl.pallas_call(
        _mlp3_body,
        grid=(m // blk,),
        in_specs=[pl.BlockSpec((blk, d), lambda i: (i, 0)),
                  full(w1), full(b1), full(w2), full(b2), full(w3), full(b3)],
        out_specs=pl.BlockSpec((blk, dout), lambda i: (i, 0)),
        out_shape=jax.ShapeDtypeStruct((m, dout), jnp.float32),
    )(x, w1, b1, w2, b2, w3, b3)


def _edge_body(e_ref, s_ref, r_ref, g_ref, w1e, w1s, w1r, w1g, b1, w2, b2,
               w3, b3, gam, bet, ne_ref, eo_ref):
    c1 = g_ref[0:1, :] @ w1g[...] + b1[...]
    h = jax.nn.gelu(e_ref[...] @ w1e[...] + s_ref[...] @ w1s[...]
                    + r_ref[...] @ w1r[...] + c1)
    h = jax.nn.gelu(h @ w2[...] + b2[...])
    ne = h @ w3[...] + b3[...]
    ne_ref[...] = ne
    if eo_ref is not None:
        eo_ref[...] = _layer_norm(e_ref[...] + ne, gam[...], bet[...])


def _edge_step(e, sent, recv, g8, p, gam, bet, want_next):
    (w1, b1), (w2, b2), (w3, b3) = p
    w1e, w1s, w1r, w1g = w1[:_L], w1[_L:2 * _L], w1[2 * _L:3 * _L], w1[3 * _L:]
    full = lambda a: pl.BlockSpec(a.shape, lambda i: (0,) * a.ndim)
    row = pl.BlockSpec((_BLKE, _L), lambda i: (i, 0))
    b1, b2, b3 = b1.reshape(1, -1), b2.reshape(1, -1), b3.reshape(1, -1)
    if want_next:
        body = _edge_body
        out_specs = (row, row)
        out_shape = (jax.ShapeDtypeStruct((_E, _L), jnp.float32),
                     jax.ShapeDtypeStruct((_E, _L), jnp.float32))
    else:
        body = functools.partial(_edge_body, eo_ref=None)
        out_specs = (row,)
        out_shape = (jax.ShapeDtypeStruct((_E, _L), jnp.float32),)
    return pl.pallas_call(
        body,
        grid=(_E // _BLKE,),
        in_specs=[row, row, row, full(g8), full(w1e), full(w1s), full(w1r),
                  full(w1g), full(b1), full(w2), full(b2), full(w3), full(b3),
                  full(gam), full(bet)],
        out_specs=out_specs,
        out_shape=out_shape,
    )(e, sent, recv, g8, w1e, w1s, w1r, w1g, b1, w2, b2, w3, b3, gam, bet)


def _node_body(n_ref, p_ref, g_ref, w1n, w1a, w1g, b1, w2, b2, w3, b3, gam,
               bet, no_ref):
    agg = p_ref[0] + p_ref[1]
    c1 = g_ref[0:1, :] @ w1g[...] + b1[...]
    h = jax.nn.gelu(n_ref[...] @ w1n[...] + agg @ w1a[...] + c1)
    h = jax.nn.gelu(h @ w2[...] + b2[...])
    nn = h @ w3[...] + b3[...]
    no_ref[...] = _layer_norm(n_ref[...] + nn, gam[...], bet[...])


def _node_step(n, part, g8, p, gam, bet):
    (w1, b1), (w2, b2), (w3, b3) = p
    w1n, w1a, w1g = w1[:_L], w1[_L:2 * _L], w1[2 * _L:]
    full = lambda a: pl.BlockSpec(a.shape, lambda i: (0,) * a.ndim)
    row = pl.BlockSpec((_BLKN, _L), lambda i: (i, 0))
    b1, b2, b3 = b1.reshape(1, -1), b2.reshape(1, -1), b3.reshape(1, -1)
    return pl.pallas_call(
        _node_body,
        grid=(_N // _BLKN,),
        in_specs=[row, pl.BlockSpec((_NC, _BLKN, _L), lambda i: (0, i, 0)),
                  full(g8), full(w1n), full(w1a), full(w1g), full(b1),
                  full(w2), full(b2), full(w3), full(b3), full(gam),
                  full(bet)],
        out_specs=row,
        out_shape=jax.ShapeDtypeStruct((_N, _L), jnp.float32),
    )(n, part, g8, w1n, w1a, w1g, b1, w2, b2, w3, b3, gam, bet)


@functools.cache
def _sc_mesh():
    return plsc.VectorSubcoreMesh(core_axis_name="c", subcore_axis_name="s",
                                  num_cores=_NC, num_subcores=_NS)


def _gather_pair(table, sidx, ridx):
    @functools.partial(
        pl.kernel,
        out_type=(jax.ShapeDtypeStruct((_E, _L), jnp.float32),
                  jax.ShapeDtypeStruct((_E, _L), jnp.float32)),
        mesh=_sc_mesh(),
        compiler_params=pltpu.CompilerParams(use_tc_tiling_on_sc=False),
        scratch_types=[
            pltpu.VMEM((_CH,), jnp.int32),
            pltpu.VMEM((_CH,), jnp.int32),
            pltpu.VMEM((_CH, _L), jnp.float32),
            pltpu.VMEM((_CH, _L), jnp.float32),
            pltpu.SemaphoreType.DMA,
            pltpu.SemaphoreType.DMA,
        ])
    def k(tab, sh, rh, os_, or_, siv, riv, srv, rrv, ss, sr):
        wid = lax.axis_index("c") * _NS + lax.axis_index("s")

        def chunk(j):
            base = j * _CH
            pltpu.sync_copy(sh.at[pl.ds(base, _CH)], siv)
            pltpu.sync_copy(rh.at[pl.ds(base, _CH)], riv)
            a = pltpu.async_copy(tab.at[siv], srv, ss)
            b = pltpu.async_copy(tab.at[riv], rrv, sr)
            a.wait()
            b.wait()
            pltpu.sync_copy(srv, os_.at[pl.ds(base, _CH)])
            pltpu.sync_copy(rrv, or_.at[pl.ds(base, _CH)])

        def body(i, _):
            chunk(i * _NW + wid)
            return 0

        lax.fori_loop(0, _FULL, body, 0)

        @pl.when(wid < _REM)
        def _():
            chunk(_FULL * _NW + wid)

    return k(table, sidx, ridx)


def _scatter_partials(vals, ridx):
    @functools.partial(
        pl.kernel,
        out_type=jax.ShapeDtypeStruct((_NC, _N, _L), jnp.float32),
        mesh=_sc_mesh(),
        compiler_params=pltpu.CompilerParams(use_tc_tiling_on_sc=False),
        scratch_types=[
            pltpu.VMEM((_CH,), jnp.int32),
            pltpu.VMEM((_CH, _L), jnp.float32),
            pltpu.VMEM((_ZR, _L), jnp.float32),
            pltpu.VMEM_SHARED((_N, _L), jnp.float32),
        ])
    def k(vh, rh, oh, riv, vrv, zb, acc):
        cid = lax.axis_index("c")
        sid = lax.axis_index("s")
        wid = cid * _NS + sid

        def zbody(i, _):
            zb[i // 4, pl.ds((i % 4) * 16, 16)] = jnp.zeros((16,), jnp.float32)
            return 0

        lax.fori_loop(0, _ZR * 4, zbody, 0)
        for z in range(_RPS // _ZR):
            pltpu.sync_copy(zb, acc.at[pl.ds(sid * _RPS + z * _ZR, _ZR)])
        plsc.subcore_barrier()

        def chunk(j):
            base = j * _CH
            pltpu.sync_copy(rh.at[pl.ds(base, _CH)], riv)
            pltpu.sync_copy(vh.at[pl.ds(base, _CH)], vrv)
            pltpu.sync_copy(vrv, acc.at[riv], add=True)

        def body(i, _):
            chunk(i * _NW + wid)
            return 0

        lax.fori_loop(0, _FULL, body, 0)

        @pl.when(wid < _REM)
        def _():
            chunk(_FULL * _NW + wid)

        plsc.subcore_barrier()
        pltpu.sync_copy(acc.at[pl.ds(sid * _RPS, _RPS)],
                        oh.at[cid, pl.ds(sid * _RPS, _RPS)])

    return k(vals, ridx)


def kernel(nodes, edges, globals_, senders, receivers, params,
           deterministic=True):
    senders = senders.astype(jnp.int32)
    receivers = receivers.astype(jnp.int32)
    g8 = jnp.broadcast_to(globals_.astype(jnp.float32),
                          (8, globals_.shape[-1]))
    gam = params['ln_gamma'].reshape(1, -1)
    bet = params['ln_beta'].reshape(1, -1)

    n = _mlp3(nodes, params['node_embed'], _BLKN)
    e = _mlp3(edges, params['edge_embed'], _BLKE)
    for s in range(3):
        sent, recv = _gather_pair(n, senders, receivers)
        if s < 2:
            new_e, e = _edge_step(e, sent, recv, g8, params['edge_upd'][s],
                                  gam, bet, True)
        else:
            (new_e,) = _edge_step(e, sent, recv, g8, params['edge_upd'][s],
                                  gam, bet, False)
        part = _scatter_partials(new_e, receivers)
        n = _node_step(n, part, g8, params['node_upd'][s], gam, bet)
    return _mlp3(n, params['decoder'], _BLKN)


# speedup vs baseline: 2.1116x; 1.0849x over previous
import functools

import jax
import jax.numpy as jnp
from jax import lax
from jax.experimental import pallas as pl
from jax.experimental.pallas import tpu as pltpu
from jax.experimental.pallas import tpu_sc as plsc

_N = 10000
_E = 320000
_L = 64

_NC = 2
_NS = 16
_NW = _NC * _NS
_CH = 128
_NCHUNKS = _E // _CH
_FULL = _NCHUNKS // _NW
_REM = _NCHUNKS - _FULL * _NW
_RPS = _N // _NS
_ZR = 125

_BLKE = 2560
_BLKN = 2000

_EPS = 1e-6


def _layer_norm(x, gam, bet):
    mu = jnp.mean(x, axis=-1, keepdims=True)
    var = jnp.mean(jnp.square(x - mu), axis=-1, keepdims=True)
    return gam * ((x - mu) / jnp.sqrt(var + _EPS)) + bet


def _mlp3_body(x_ref, w1, b1, w2, b2, w3, b3, o_ref):
    h = jax.nn.gelu(x_ref[...] @ w1[...] + b1[...])
    h = jax.nn.gelu(h @ w2[...] + b2[...])
    o_ref[...] = h @ w3[...] + b3[...]


def _mlp3(x, p, blk):
    (w1, b1), (w2, b2), (w3, b3) = p
    m, d = x.shape
    dout = w3.shape[1]
    full = lambda a: pl.BlockSpec(a.shape, lambda i: (0,) * a.ndim)
    b1, b2, b3 = b1.reshape(1, -1), b2.reshape(1, -1), b3.reshape(1, -1)
    return pl.pallas_call(
        _mlp3_body,
        grid=(m // blk,),
        in_specs=[pl.BlockSpec((blk, d), lambda i: (i, 0)),
                  full(w1), full(b1), full(w2), full(b2), full(w3), full(b3)],
        out_specs=pl.BlockSpec((blk, dout), lambda i: (i, 0)),
        out_shape=jax.ShapeDtypeStruct((m, dout), jnp.float32),
    )(x, w1, b1, w2, b2, w3, b3)


def _edge_body(e_ref, s_ref, r_ref, g_ref, w1e, w1s, w1r, w1g, b1, w2, b2,
               w3, b3, gam, bet, ne_ref, eo_ref):
    c1 = g_ref[0:1, :] @ w1g[...] + b1[...]
    h = jax.nn.gelu(e_ref[...] @ w1e[...] + s_ref[...] @ w1s[...]
                    + r_ref[...] @ w1r[...] + c1)
    h = jax.nn.gelu(h @ w2[...] + b2[...])
    ne = h @ w3[...] + b3[...]
    ne_ref[...] = ne
    if eo_ref is not None:
        eo_ref[...] = _layer_norm(e_ref[...] + ne, gam[...], bet[...])


def _edge_step(e, sent, recv, g8, p, gam, bet, want_next):
    (w1, b1), (w2, b2), (w3, b3) = p
    w1e, w1s, w1r, w1g = w1[:_L], w1[_L:2 * _L], w1[2 * _L:3 * _L], w1[3 * _L:]
    full = lambda a: pl.BlockSpec(a.shape, lambda i: (0,) * a.ndim)
    row = pl.BlockSpec((_BLKE, _L), lambda i: (i, 0))
    b1, b2, b3 = b1.reshape(1, -1), b2.reshape(1, -1), b3.reshape(1, -1)
    if want_next:
        body = _edge_body
        out_specs = (row, row)
        out_shape = (jax.ShapeDtypeStruct((_E, _L), jnp.float32),
                     jax.ShapeDtypeStruct((_E, _L), jnp.float32))
    else:
        body = functools.partial(_edge_body, eo_ref=None)
        out_specs = (row,)
        out_shape = (jax.ShapeDtypeStruct((_E, _L), jnp.float32),)
    return pl.pallas_call(
        body,
        grid=(_E // _BLKE,),
        in_specs=[row, row, row, full(g8), full(w1e), full(w1s), full(w1r),
                  full(w1g), full(b1), full(w2), full(b2), full(w3), full(b3),
                  full(gam), full(bet)],
        out_specs=out_specs,
        out_shape=out_shape,
    )(e, sent, recv, g8, w1e, w1s, w1r, w1g, b1, w2, b2, w3, b3, gam, bet)


def _node_body(n_ref, p_ref, g_ref, w1n, w1a, w1g, b1, w2, b2, w3, b3, gam,
               bet, no_ref):
    agg = p_ref[0] + p_ref[1]
    c1 = g_ref[0:1, :] @ w1g[...] + b1[...]
    h = jax.nn.gelu(n_ref[...] @ w1n[...] + agg @ w1a[...] + c1)
    h = jax.nn.gelu(h @ w2[...] + b2[...])
    nn = h @ w3[...] + b3[...]
    no_ref[...] = _layer_norm(n_ref[...] + nn, gam[...], bet[...])


def _node_step(n, part, g8, p, gam, bet):
    (w1, b1), (w2, b2), (w3, b3) = p
    w1n, w1a, w1g = w1[:_L], w1[_L:2 * _L], w1[2 * _L:]
    full = lambda a: pl.BlockSpec(a.shape, lambda i: (0,) * a.ndim)
    row = pl.BlockSpec((_BLKN, _L), lambda i: (i, 0))
    b1, b2, b3 = b1.reshape(1, -1), b2.reshape(1, -1), b3.reshape(1, -1)
    return pl.pallas_call(
        _node_body,
        grid=(_N // _BLKN,),
        in_specs=[row, pl.BlockSpec((_NC, _BLKN, _L), lambda i: (0, i, 0)),
                  full(g8), full(w1n), full(w1a), full(w1g), full(b1),
                  full(w2), full(b2), full(w3), full(b3), full(gam),
                  full(bet)],
        out_specs=row,
        out_shape=jax.ShapeDtypeStruct((_N, _L), jnp.float32),
    )(n, part, g8, w1n, w1a, w1g, b1, w2, b2, w3, b3, gam, bet)


@functools.cache
def _sc_mesh():
    return plsc.VectorSubcoreMesh(core_axis_name="c", subcore_axis_name="s",
                                  num_cores=_NC, num_subcores=_NS)


def _gather_pair(table, sidx, ridx):
    @functools.partial(
        pl.kernel,
        out_type=(jax.ShapeDtypeStruct((_E, _L), jnp.float32),
                  jax.ShapeDtypeStruct((_E, _L), jnp.float32)),
        mesh=_sc_mesh(),
        compiler_params=pltpu.CompilerParams(use_tc_tiling_on_sc=False),
        scratch_types=[
            pltpu.VMEM((2, _CH), jnp.int32),
            pltpu.VMEM((2, _CH), jnp.int32),
            pltpu.VMEM((2, _CH, _L), jnp.float32),
            pltpu.VMEM((2, _CH, _L), jnp.float32),
        ] + [pltpu.SemaphoreType.DMA] * 8)
    def k(tab, sh, rh, os_, or_, siv, riv, srv, rrv,
          gs0, gr0, ws0, wr0, gs1, gr1, ws1, wr1):
        wid = lax.axis_index("c") * _NS + lax.axis_index("s")
        sems = ((gs0, gr0, ws0, wr0), (gs1, gr1, ws1, wr1))

        def start(j, b):
            base = j * _CH
            pltpu.sync_copy(sh.at[pl.ds(base, _CH)], siv.at[b])
            pltpu.sync_copy(rh.at[pl.ds(base, _CH)], riv.at[b])
            pltpu.async_copy(tab.at[siv.at[b]], srv.at[b], sems[b][0])
            pltpu.async_copy(tab.at[riv.at[b]], rrv.at[b], sems[b][1])

        def drain(j, b):
            base = j * _CH
            pltpu.make_async_copy(tab.at[siv.at[b]], srv.at[b],
                                  sems[b][0]).wait()
            pltpu.make_async_copy(tab.at[riv.at[b]], rrv.at[b],
                                  sems[b][1]).wait()
            pltpu.async_copy(srv.at[b], os_.at[pl.ds(base, _CH)], sems[b][2])
            pltpu.async_copy(rrv.at[b], or_.at[pl.ds(base, _CH)], sems[b][3])

        def wait_writes(j, b):
            base = j * _CH
            pltpu.make_async_copy(srv.at[b], os_.at[pl.ds(base, _CH)],
                                  sems[b][2]).wait()
            pltpu.make_async_copy(rrv.at[b], or_.at[pl.ds(base, _CH)],
                                  sems[b][3]).wait()

        def body(i, _):
            for b in range(2):
                j = (2 * i + b) * _NW + wid

                @pl.when(i > 0)
                def _():
                    wait_writes(j - 2 * _NW, b)

                start(j, b)
                drain(j, b)
            return 0

        lax.fori_loop(0, _FULL // 2, body, 0)
        for b in range(2):
            wait_writes((_FULL - 2 + b) * _NW + wid, b)

        @pl.when(wid < _REM)
        def _():
            j = _FULL * _NW + wid
            start(j, 0)
            drain(j, 0)
            wait_writes(j, 0)

    return k(table, sidx, ridx)


def _scatter_partials(vals, ridx):
    @functools.partial(
        pl.kernel,
        out_type=jax.ShapeDtypeStruct((_NC, _N, _L), jnp.float32),
        mesh=_sc_mesh(),
        compiler_params=pltpu.CompilerParams(use_tc_tiling_on_sc=False),
        scratch_types=[
            pltpu.VMEM((2, _CH), jnp.int32),
            pltpu.VMEM((2, _CH, _L), jnp.float32),
            pltpu.VMEM((_ZR, _L), jnp.float32),
            pltpu.VMEM_SHARED((_N, _L), jnp.float32),
        ] + [pltpu.SemaphoreType.DMA] * 6)
    def k(vh, rh, oh, riv, vrv, zb, acc, li0, lv0, sa0, li1, lv1, sa1):
        cid = lax.axis_index("c")
        sid = lax.axis_index("s")
        wid = cid * _NS + sid
        sems = ((li0, lv0, sa0), (li1, lv1, sa1))

        def zbody(i, _):
            zb[i // 4, pl.ds((i % 4) * 16, 16)] = jnp.zeros((16,), jnp.float32)
            return 0

        lax.fori_loop(0, _ZR * 4, zbody, 0)
        for z in range(_RPS // _ZR):
            pltpu.sync_copy(zb, acc.at[pl.ds(sid * _RPS + z * _ZR, _ZR)])
        plsc.subcore_barrier()

        def load(j, b):
            base = j * _CH
            pltpu.async_copy(rh.at[pl.ds(base, _CH)], riv.at[b], sems[b][0])
            pltpu.async_copy(vh.at[pl.ds(base, _CH)], vrv.at[b], sems[b][1])

        def scat(j, b):
            base = j * _CH
            pltpu.make_async_copy(rh.at[pl.ds(base, _CH)], riv.at[b],
                                  sems[b][0]).wait()
            pltpu.make_async_copy(vh.at[pl.ds(base, _CH)], vrv.at[b],
                                  sems[b][1]).wait()
            pltpu.async_copy(vrv.at[b], acc.at[riv.at[b]], sems[b][2],
                             add=True)

        def wait_scat(b):
            pltpu.make_async_copy(vrv.at[b], acc.at[riv.at[b]],
                                  sems[b][2]).wait()

        def body(i, _):
            for b in range(2):
                j = (2 * i + b) * _NW + wid

                @pl.when(i > 0)
                def _():
                    wait_scat(b)

                load(j, b)
                scat(j, b)
            return 0

        lax.fori_loop(0, _FULL // 2, body, 0)
        for b in range(2):
            wait_scat(b)

        @pl.when(wid < _REM)
        def _():
            j = _FULL * _NW + wid
            load(j, 0)
            scat(j, 0)
            wait_scat(0)

        plsc.subcore_barrier()
        pltpu.sync_copy(acc.at[pl.ds(sid * _RPS, _RPS)],
                        oh.at[cid, pl.ds(sid * _RPS, _RPS)])

    return k(vals, ridx)


def kernel(nodes, edges, globals_, senders, receivers, params,
           deterministic=True):
    senders = senders.astype(jnp.int32)
    receivers = receivers.astype(jnp.int32)
    g8 = jnp.broadcast_to(globals_.astype(jnp.float32),
                          (8, globals_.shape[-1]))
    gam = params['ln_gamma'].reshape(1, -1)
    bet = params['ln_beta'].reshape(1, -1)

    n = _mlp3(nodes, params['node_embed'], _BLKN)
    e = _mlp3(edges, params['edge_embed'], _BLKE)
    for s in range(3):
        sent, recv = _gather_pair(n, senders, receivers)
        if s < 2:
            new_e, e = _edge_step(e, sent, recv, g8, params['edge_upd'][s],
                                  gam, bet, True)
        else:
            (new_e,) = _edge_step(e, sent, recv, g8, params['edge_upd'][s],
                                  gam, bet, False)
        part = _scatter_partials(new_e, receivers)
        n = _node_step(n, part, g8, params['node_upd'][s], gam, bet)
    return _mlp3(n, params['decoder'], _BLKN)


# speedup vs baseline: 2.8310x; 1.3407x over previous
import functools

import jax
import jax.numpy as jnp
from jax import lax
from jax.experimental import pallas as pl
from jax.experimental.pallas import tpu as pltpu
from jax.experimental.pallas import tpu_sc as plsc

_N = 10000
_E = 320000
_L = 64

_NC = 2
_NS = 16
_NW = _NC * _NS
_CH = 128
_NCHUNKS = _E // _CH
_FULL = _NCHUNKS // _NW
_REM = _NCHUNKS - _FULL * _NW
_RPS = _N // _NS
_ZR = 125

_E2 = _E // 2
_BLKE = 2560
_BLKN = 2000

_EPS = 1e-6


def _layer_norm(x, gam, bet):
    mu = jnp.mean(x, axis=-1, keepdims=True)
    var = jnp.mean(jnp.square(x - mu), axis=-1, keepdims=True)
    return gam * ((x - mu) / jnp.sqrt(var + _EPS)) + bet


def _mlp3_body(x_ref, w1, b1, w2, b2, w3, b3, o_ref):
    h = jax.nn.gelu(x_ref[...] @ w1[...] + b1[...])
    h = jax.nn.gelu(h @ w2[...] + b2[...])
    o_ref[...] = h @ w3[...] + b3[...]


def _mlp3(x, p, blk):
    (w1, b1), (w2, b2), (w3, b3) = p
    m, d = x.shape
    dout = w3.shape[1]
    full = lambda a: pl.BlockSpec(a.shape, lambda i: (0,) * a.ndim)
    b1, b2, b3 = b1.reshape(1, -1), b2.reshape(1, -1), b3.reshape(1, -1)
    return pl.pallas_call(
        _mlp3_body,
        grid=(m // blk,),
        in_specs=[pl.BlockSpec((blk, d), lambda i: (i, 0)),
                  full(w1), full(b1), full(w2), full(b2), full(w3), full(b3)],
        out_specs=pl.BlockSpec((blk, dout), lambda i: (i, 0)),
        out_shape=jax.ShapeDtypeStruct((m, dout), jnp.float32),
    )(x, w1, b1, w2, b2, w3, b3)


def _edge_body(e_ref, sr_ref, g_ref, w1e, w1s, w1r, w1g, b1, w2, b2,
               w3, b3, gam, bet, ne_ref, eo_ref):
    sent = sr_ref[:, :_L]
    recv = sr_ref[:, _L:]
    c1 = g_ref[0:1, :] @ w1g[...] + b1[...]
    h = jax.nn.gelu(e_ref[...] @ w1e[...] + sent @ w1s[...]
                    + recv @ w1r[...] + c1)
    h = jax.nn.gelu(h @ w2[...] + b2[...])
    ne = h @ w3[...] + b3[...]
    ne_ref[...] = ne
    if eo_ref is not None:
        eo_ref[...] = _layer_norm(e_ref[...] + ne, gam[...], bet[...])


def _edge_step(e, sentrecv, g8, p, gam, bet, want_next):
    (w1, b1), (w2, b2), (w3, b3) = p
    w1e, w1s, w1r, w1g = w1[:_L], w1[_L:2 * _L], w1[2 * _L:3 * _L], w1[3 * _L:]
    full = lambda a: pl.BlockSpec(a.shape, lambda i: (0,) * a.ndim)
    row = pl.BlockSpec((_BLKE, _L), lambda i: (i, 0))
    wrow = pl.BlockSpec((_BLKE, 2 * _L), lambda i: (i, 0))
    b1, b2, b3 = b1.reshape(1, -1), b2.reshape(1, -1), b3.reshape(1, -1)
    if want_next:
        body = _edge_body
        out_specs = (row, row)
        out_shape = (jax.ShapeDtypeStruct((_E, _L), jnp.float32),
                     jax.ShapeDtypeStruct((_E, _L), jnp.float32))
    else:
        body = functools.partial(_edge_body, eo_ref=None)
        out_specs = (row,)
        out_shape = (jax.ShapeDtypeStruct((_E, _L), jnp.float32),)
    return pl.pallas_call(
        body,
        grid=(_E // _BLKE,),
        in_specs=[row, wrow, full(g8), full(w1e), full(w1s), full(w1r),
                  full(w1g), full(b1), full(w2), full(b2), full(w3), full(b3),
                  full(gam), full(bet)],
        out_specs=out_specs,
        out_shape=out_shape,
    )(e, sentrecv, g8, w1e, w1s, w1r, w1g, b1, w2, b2, w3, b3, gam, bet)


def _node_body(n_ref, p_ref, g_ref, w1n, w1a, w1g, b1, w2, b2, w3, b3, gam,
               bet, no_ref):
    agg = p_ref[0] + p_ref[1]
    c1 = g_ref[0:1, :] @ w1g[...] + b1[...]
    h = jax.nn.gelu(n_ref[...] @ w1n[...] + agg @ w1a[...] + c1)
    h = jax.nn.gelu(h @ w2[...] + b2[...])
    nn = h @ w3[...] + b3[...]
    no_ref[...] = _layer_norm(n_ref[...] + nn, gam[...], bet[...])


def _node_step(n, part, g8, p, gam, bet):
    (w1, b1), (w2, b2), (w3, b3) = p
    w1n, w1a, w1g = w1[:_L], w1[_L:2 * _L], w1[2 * _L:]
    full = lambda a: pl.BlockSpec(a.shape, lambda i: (0,) * a.ndim)
    row = pl.BlockSpec((_BLKN, _L), lambda i: (i, 0))
    b1, b2, b3 = b1.reshape(1, -1), b2.reshape(1, -1), b3.reshape(1, -1)
    return pl.pallas_call(
        _node_body,
        grid=(_N // _BLKN,),
        in_specs=[row, pl.BlockSpec((_NC, _BLKN, _L), lambda i: (0, i, 0)),
                  full(g8), full(w1n), full(w1a), full(w1g), full(b1),
                  full(w2), full(b2), full(w3), full(b3), full(gam),
                  full(bet)],
        out_specs=row,
        out_shape=jax.ShapeDtypeStruct((_N, _L), jnp.float32),
    )(n, part, g8, w1n, w1a, w1g, b1, w2, b2, w3, b3, gam, bet)


@functools.cache
def _sc_mesh():
    return plsc.VectorSubcoreMesh(core_axis_name="c", subcore_axis_name="s",
                                  num_cores=_NC, num_subcores=_NS)


def _gather_pair(table, sidx, ridx):
    @functools.partial(
        pl.kernel,
        out_type=jax.ShapeDtypeStruct((_E, 2 * _L), jnp.float32),
        mesh=_sc_mesh(),
        compiler_params=pltpu.CompilerParams(use_tc_tiling_on_sc=False),
        scratch_types=[
            pltpu.VMEM((2, _CH), jnp.int32),
            pltpu.VMEM((2, _CH), jnp.int32),
            pltpu.VMEM((2, _CH, _L), jnp.float32),
            pltpu.VMEM((2, _CH, _L), jnp.float32),
        ] + [pltpu.SemaphoreType.DMA] * 8)
    def k(tab, sh, rh, oc, siv, riv, srv, rrv,
          gs0, gr0, ws0, wr0, gs1, gr1, ws1, wr1):
        wid = lax.axis_index("c") * _NS + lax.axis_index("s")
        sems = ((gs0, gr0, ws0, wr0), (gs1, gr1, ws1, wr1))

        def start(j, b):
            base = j * _CH
            pltpu.sync_copy(sh.at[pl.ds(base, _CH)], siv.at[b])
            pltpu.sync_copy(rh.at[pl.ds(base, _CH)], riv.at[b])
            pltpu.async_copy(tab.at[siv.at[b]], srv.at[b], sems[b][0])
            pltpu.async_copy(tab.at[riv.at[b]], rrv.at[b], sems[b][1])

        def drain(j, b):
            base = j * _CH
            pltpu.make_async_copy(tab.at[siv.at[b]], srv.at[b],
                                  sems[b][0]).wait()
            pltpu.make_async_copy(tab.at[riv.at[b]], rrv.at[b],
                                  sems[b][1]).wait()
            pltpu.async_copy(srv.at[b], oc.at[pl.ds(base, _CH), pl.ds(0, _L)],
                             sems[b][2])
            pltpu.async_copy(rrv.at[b], oc.at[pl.ds(base, _CH), pl.ds(_L, _L)],
                             sems[b][3])

        def wait_writes(j, b):
            base = j * _CH
            pltpu.make_async_copy(srv.at[b],
                                  oc.at[pl.ds(base, _CH), pl.ds(0, _L)],
                                  sems[b][2]).wait()
            pltpu.make_async_copy(rrv.at[b],
                                  oc.at[pl.ds(base, _CH), pl.ds(_L, _L)],
                                  sems[b][3]).wait()

        def body(i, _):
            for b in range(2):
                j = (2 * i + b) * _NW + wid

                @pl.when(i > 0)
                def _():
                    wait_writes(j - 2 * _NW, b)

                start(j, b)
                drain(j, b)
            return 0

        lax.fori_loop(0, _FULL // 2, body, 0)
        for b in range(2):
            wait_writes((_FULL - 2 + b) * _NW + wid, b)

        @pl.when(wid < _REM)
        def _():
            j = _FULL * _NW + wid
            start(j, 0)
            drain(j, 0)
            wait_writes(j, 0)

    return k(table, sidx, ridx)


def _scatter_partials(vals, ridx):
    @functools.partial(
        pl.kernel,
        out_type=jax.ShapeDtypeStruct((_NC, _N, _L), jnp.float32),
        mesh=_sc_mesh(),
        compiler_params=pltpu.CompilerParams(use_tc_tiling_on_sc=False),
        scratch_types=[
            pltpu.VMEM((2, _CH), jnp.int32),
            pltpu.VMEM((2, _CH, _L), jnp.float32),
            pltpu.VMEM((_ZR, _L), jnp.float32),
            pltpu.VMEM_SHARED((_N, _L), jnp.float32),
        ] + [pltpu.SemaphoreType.DMA] * 6)
    def k(vh, rh, oh, riv, vrv, zb, acc, li0, lv0, sa0, li1, lv1, sa1):
        cid = lax.axis_index("c")
        sid = lax.axis_index("s")
        wid = cid * _NS + sid
        sems = ((li0, lv0, sa0), (li1, lv1, sa1))

        def zbody(i, _):
            zb[i // 4, pl.ds((i % 4) * 16, 16)] = jnp.zeros((16,), jnp.float32)
            return 0

        lax.fori_loop(0, _ZR * 4, zbody, 0)
        for z in range(_RPS // _ZR):
            pltpu.sync_copy(zb, acc.at[pl.ds(sid * _RPS + z * _ZR, _ZR)])
        plsc.subcore_barrier()

        def load(j, b):
            base = j * _CH
            pltpu.async_copy(rh.at[pl.ds(base, _CH)], riv.at[b], sems[b][0])
            pltpu.async_copy(vh.at[pl.ds(base, _CH)], vrv.at[b], sems[b][1])

        def scat(j, b):
            base = j * _CH
            pltpu.make_async_copy(rh.at[pl.ds(base, _CH)], riv.at[b],
                                  sems[b][0]).wait()
            pltpu.make_async_copy(vh.at[pl.ds(base, _CH)], vrv.at[b],
                                  sems[b][1]).wait()
            pltpu.async_copy(vrv.at[b], acc.at[riv.at[b]], sems[b][2],
                             add=True)

        def wait_scat(b):
            pltpu.make_async_copy(vrv.at[b], acc.at[riv.at[b]],
                                  sems[b][2]).wait()

        def body(i, _):
            for b in range(2):
                j = (2 * i + b) * _NW + wid

                @pl.when(i > 0)
                def _():
                    wait_scat(b)

                load(j, b)
                scat(j, b)
            return 0

        lax.fori_loop(0, _FULL // 2, body, 0)
        for b in range(2):
            wait_scat(b)

        @pl.when(wid < _REM)
        def _():
            j = _FULL * _NW + wid
            load(j, 0)
            scat(j, 0)
            wait_scat(0)

        plsc.subcore_barrier()
        pltpu.sync_copy(acc.at[pl.ds(sid * _RPS, _RPS)],
                        oh.at[cid, pl.ds(sid * _RPS, _RPS)])

    return k(vals, ridx)


def kernel(nodes, edges, globals_, senders, receivers, params,
           deterministic=True):
    senders = senders.astype(jnp.int32)
    receivers = receivers.astype(jnp.int32)
    g8 = jnp.broadcast_to(globals_.astype(jnp.float32),
                          (8, globals_.shape[-1]))
    gam = params['ln_gamma'].reshape(1, -1)
    bet = params['ln_beta'].reshape(1, -1)

    n = _mlp3(nodes, params['node_embed'], _BLKN)
    e = _mlp3(edges, params['edge_embed'], _BLKE)
    for s in range(3):
        sentrecv = _gather_pair(n, senders, receivers)
        if s < 2:
            new_e, e = _edge_step(e, sentrecv, g8, params['edge_upd'][s],
                                  gam, bet, True)
        else:
            (new_e,) = _edge_step(e, sentrecv, g8, params['edge_upd'][s],
                                  gam, bet, False)
        part = _scatter_partials(new_e, receivers)
        n = _node_step(n, part, g8, params['node_upd'][s], gam, bet)
    return _mlp3(n, params['decoder'], _BLKN)
